# baseline (speedup 1.0000x reference)
import functools

import jax
import jax.numpy as jnp
from jax import lax
from jax.experimental import pallas as pl
from jax.experimental.pallas import tpu as pltpu
from jax.experimental.pallas import tpu_sc as plsc

_BLK_ROWS = 256
_K = 8192
_NC = 2
_NS = 16
_NW = _NC * _NS
_SC_ROWS = 1024
_RPW = _SC_ROWS // _NW
_CH = 8


def _softmax_block(x_ref, o_ref):
    x = x_ref[...]
    m = jnp.max(x, axis=-1, keepdims=True)
    e = jnp.exp(x - m)
    s = jnp.sum(e, axis=-1, keepdims=True)
    o_ref[...] = e * (1.0 / s)


def _tc_softmax(x):
    rows = x.shape[0]
    return pl.pallas_call(
        _softmax_block,
        grid=(rows // _BLK_ROWS,),
        in_specs=[pl.BlockSpec((_BLK_ROWS, _K), lambda i: (i, 0))],
        out_specs=pl.BlockSpec((_BLK_ROWS, _K), lambda i: (i, 0)),
        out_shape=jax.ShapeDtypeStruct((rows, _K), x.dtype),
        compiler_params=pltpu.CompilerParams(
            dimension_semantics=("arbitrary",),
        ),
    )(x)


@functools.partial(
    pl.kernel,
    out_type=jax.ShapeDtypeStruct((_SC_ROWS, _K), jnp.float32),
    mesh=plsc.VectorSubcoreMesh(core_axis_name="c", subcore_axis_name="s"),
    scratch_types=[pltpu.VMEM((_CH, _K), jnp.float32)],
)
def _sc_copy(x_hbm, o_hbm, buf):
    wid = lax.axis_index("s") * _NC + lax.axis_index("c")
    base = wid * _RPW

    def step(i, carry):
        off = base + i * _CH
        pltpu.sync_copy(x_hbm.at[pl.ds(off, _CH)], buf)
        pltpu.sync_copy(buf, o_hbm.at[pl.ds(off, _CH)])
        return carry

    lax.fori_loop(0, _RPW // _CH, step, 0)


def kernel(scores):
    b, h, q, k = scores.shape
    rows = b * h * q
    x = scores.reshape(rows, k)
    out_tc = _tc_softmax(x[: rows - _SC_ROWS])
    out_sc = _sc_copy(x[rows - _SC_ROWS :])
    return jnp.concatenate([out_tc, out_sc], axis=0).reshape(b, h, q, k)


# speedup vs baseline: 3.1577x; 3.1577x over previous
import jax
import jax.numpy as jnp
from jax.experimental import pallas as pl
from jax.experimental.pallas import tpu as pltpu

_BLK_ROWS = 256


def _softmax_block(x_ref, o_ref):
    x = x_ref[...]
    m = jnp.max(x, axis=-1, keepdims=True)
    e = jnp.exp(x - m)
    s = jnp.sum(e, axis=-1, keepdims=True)
    o_ref[...] = e * (1.0 / s)


def kernel(scores):
    b, h, q, k = scores.shape
    rows = b * h * q
    x = scores.reshape(rows, k)
    out = pl.pallas_call(
        _softmax_block,
        grid=(rows // _BLK_ROWS,),
        in_specs=[pl.BlockSpec((_BLK_ROWS, k), lambda i: (i, 0))],
        out_specs=pl.BlockSpec((_BLK_ROWS, k), lambda i: (i, 0)),
        out_shape=jax.ShapeDtypeStruct((rows, k), scores.dtype),
        compiler_params=pltpu.CompilerParams(
            dimension_semantics=("parallel",),
        ),
    )(x)
    return out.reshape(b, h, q, k)


# speedup vs baseline: 3.1607x; 1.0010x over previous
import jax
import jax.numpy as jnp
from jax.experimental import pallas as pl
from jax.experimental.pallas import tpu as pltpu

_BLK_ROWS = 256


def _softmax_block(x_ref, o_ref):
    m = jnp.max(x_ref[...], axis=-1, keepdims=True)
    o_ref[...] = jnp.exp(x_ref[...] - m)
    e = o_ref[...]
    s = jnp.sum(e, axis=-1, keepdims=True)
    o_ref[...] = e * (1.0 / s)


def kernel(scores):
    b, h, q, k = scores.shape
    rows = b * h * q
    x = scores.reshape(rows, k)
    out = pl.pallas_call(
        _softmax_block,
        grid=(rows // _BLK_ROWS,),
        in_specs=[pl.BlockSpec((_BLK_ROWS, k), lambda i: (i, 0))],
        out_specs=pl.BlockSpec((_BLK_ROWS, k), lambda i: (i, 0)),
        out_shape=jax.ShapeDtypeStruct((rows, k), scores.dtype),
        compiler_params=pltpu.CompilerParams(
            dimension_semantics=("parallel",),
        ),
    )(x)
    return out.reshape(b, h, q, k)
